# baseline (speedup 1.0000x reference)
import jax
import jax.numpy as jnp
from jax.experimental import pallas as pl
from jax.experimental.pallas import tpu as pltpu

_B, _S, _D, _H = 1, 2048, 768, 12
_EA, _EF, _K = 8, 8, 2
_DH = _D // _H
_DFF = 4 * _D
_SB_C = 512
_SB_F = 256
_QB = 2048


def _mm_t(a, b):
    return jax.lax.dot_general(a, b, (((1,), (1,)), ((), ())),
                               preferred_element_type=jnp.float32)


def _router_a_body(x_ref, mrw_ref, mrb_ref, ar_ref, ti_ref, awk_ref):
    xm = jnp.mean(x_ref[...], axis=0, keepdims=True)
    logits = jnp.dot(xm, mrw_ref[...],
                     preferred_element_type=jnp.float32) + mrb_ref[...]
    ar = jax.nn.softmax(logits, axis=-1)
    iota = jax.lax.broadcasted_iota(jnp.int32, (1, _EA), 1)
    i1 = jnp.argmax(ar, axis=-1)
    m1 = iota == i1[:, None]
    i2 = jnp.argmax(jnp.where(m1, -1.0, ar), axis=-1)
    m2 = iota == i2[:, None]
    mask = (m1 | m2).astype(jnp.float32)
    aw = ar * mask
    aw = aw / (jnp.sum(aw, axis=-1, keepdims=True) + 1e-9)
    w1 = jnp.sum(jnp.where(m1, aw, 0.0), axis=-1)
    w2 = jnp.sum(jnp.where(m2, aw, 0.0), axis=-1)
    ar_ref[...] = ar
    ti_ref[...] = jnp.stack([i1, i2], axis=-1).astype(jnp.int32)
    awk_ref[...] = jnp.stack([w1, w2], axis=-1)


def _qkv_body(ti_ref, x_ref, w_ref, b_ref, out_ref):
    del ti_ref
    y = _mm_t(x_ref[...], w_ref[0]) + b_ref[0]
    out_ref[0] = y


def _attn_body(q_ref, k_ref, v_ref, o_ref):
    log2e_over_sqrt_dh = 1.4426950408889634 / 8.0
    outs = []
    for hh in range(2):
        q = q_ref[0][:, hh * _DH:(hh + 1) * _DH] * log2e_over_sqrt_dh
        k = k_ref[0][:, hh * _DH:(hh + 1) * _DH]
        v = v_ref[0][:, hh * _DH:(hh + 1) * _DH]
        s2 = _mm_t(q, k)
        m = jnp.max(s2, axis=-1, keepdims=True)
        e2 = jnp.exp2(s2 - m)
        u = jnp.dot(e2, v, preferred_element_type=jnp.float32)
        r = 1.0 / jnp.sum(e2, axis=-1, keepdims=True)
        outs.append(u * r)
    o_ref[0] = jnp.concatenate(outs, axis=1)


def _combine_body(ti_ref, o_ref, wo_ref, bo_ref, wp_ref, bp_ref, awk_ref,
                  x_ref, g1_ref, b1_ref, out_ref, acc_ref):
    del ti_ref
    e = pl.program_id(0)
    sb = pl.program_id(1)
    t = _mm_t(o_ref[0], wo_ref[0]) + bo_ref[0]
    r = _mm_t(t, wp_ref[0]) + bp_ref[0]
    w_e = jnp.sum(jnp.where(
        jax.lax.broadcasted_iota(jnp.int32, (1, _K), 1) == e,
        awk_ref[...], 0.0))
    c = w_e * r
    sl = pl.ds(sb * _SB_C, _SB_C)

    @pl.when(e == 0)
    def _():
        acc_ref[sl, :] = c

    @pl.when(e == 1)
    def _():
        attn = acc_ref[sl, :] + c
        out_ref[...] = (g1_ref[...] * (x_ref[...] + attn)
                        / jnp.sqrt(1.0 + 1e-5) + b1_ref[...])


_TILE = 256
_NROW = -(-(2 * _S + _EF * (_TILE - 1)) // _TILE) * _TILE
_NT = _NROW // _TILE


def _meta_body(x1_ref, frw_ref, frb_ref, fr_ref, r01_ref, w01_ref, te_ref):
    logits = jnp.dot(x1_ref[...], frw_ref[...],
                     preferred_element_type=jnp.float32) + frb_ref[...]
    fr = jax.nn.softmax(logits, axis=-1)
    iota = jax.lax.broadcasted_iota(jnp.int32, (_S, _EF), 1)
    i1 = jnp.argmax(fr, axis=-1)
    m1 = iota == i1[:, None]
    i2 = jnp.argmax(jnp.where(m1, -1.0, fr), axis=-1)
    m2 = iota == i2[:, None]
    fwm = fr * (m1 | m2).astype(jnp.float32)
    fw = fwm / (jnp.sum(fwm, axis=-1, keepdims=True) + 1e-9)
    fr_ref[...] = fr

    m1f = m1.astype(jnp.float32)
    m2f = m2.astype(jnp.float32)
    rio = jax.lax.broadcasted_iota(jnp.int32, (_S, 1), 0)
    cio = jax.lax.broadcasted_iota(jnp.int32, (1, _S), 1)
    tri = (cio <= rio).astype(jnp.bfloat16)
    cs1 = jnp.dot(tri, m1.astype(jnp.bfloat16),
                  preferred_element_type=jnp.float32)
    cs2 = jnp.dot(tri, m2.astype(jnp.bfloat16),
                  preferred_element_type=jnp.float32)
    tot1 = cs1[_S - 1:_S, :]
    counts = tot1 + cs2[_S - 1:_S, :]
    padded = jnp.floor((counts + (_TILE - 1)) * (1.0 / _TILE)) * _TILE
    e8r = jax.lax.broadcasted_iota(jnp.int32, (_EF, _EF), 0)
    e8c = jax.lax.broadcasted_iota(jnp.int32, (_EF, _EF), 1)
    t8 = (e8r < e8c).astype(jnp.float32)
    offs = jnp.dot(padded, t8, preferred_element_type=jnp.float32)
    r0 = jnp.sum(m1f * (cs1 - 1.0 + offs), axis=1, keepdims=True)
    r1 = jnp.sum(m2f * (cs2 - 1.0 + offs + tot1), axis=1, keepdims=True)
    r01_ref[...] = jnp.concatenate([r0, r1], axis=1)
    w0 = jnp.sum(fw * m1f, axis=1, keepdims=True)
    w1 = jnp.sum(fw * m2f, axis=1, keepdims=True)
    w01_ref[...] = jnp.concatenate([w0, w1], axis=1)
    tpos = (jax.lax.broadcasted_iota(jnp.int32, (_NT, 1), 0)
            .astype(jnp.float32) * _TILE)
    te = jnp.sum((offs <= tpos).astype(jnp.int32), axis=1,
                 keepdims=True) - 1
    te_ref[...] = te


def _gffn_body(te_ref, r0_ref, r1_ref, x1_ref, w1_ref, b1_ref, w2_ref,
               b2_ref, yg_ref):
    del te_ref
    t = pl.program_id(0)
    rowio = (jax.lax.broadcasted_iota(jnp.int32, (_TILE, 1), 0)
             .astype(jnp.float32) + jnp.float32(_TILE) * t)
    r0v = r0_ref[0]
    r1v = r1_ref[0]
    p = ((r0v == rowio) | (r1v == rowio)).astype(jnp.float32)
    xg = jnp.dot(p, x1_ref[...], preferred_element_type=jnp.float32)
    z = _mm_t(xg, w1_ref[0]) + b1_ref[0]
    h = 0.5 * z * (1.0 + jax.lax.erf(z * (2.0 ** -0.5)))
    yg_ref[...] = _mm_t(h, w2_ref[0]) + b2_ref[0]


def _fcomb_body(yg_ref, r01_ref, w01_ref, x1_ref, g2_ref, bn2_ref, out_ref):
    r0c = r01_ref[:, 0:1]
    r1c = r01_ref[:, 1:2]
    w0c = w01_ref[:, 0:1]
    w1c = w01_ref[:, 1:2]
    colio = (jax.lax.broadcasted_iota(jnp.int32, (1, _NROW), 1)
             .astype(jnp.float32))
    g = (jnp.where(r0c == colio, w0c, 0.0)
         + jnp.where(r1c == colio, w1c, 0.0))
    acc = jnp.dot(g, yg_ref[...], preferred_element_type=jnp.float32)
    out_ref[...] = (g2_ref[...] * (x1_ref[...] + acc)
                    / jnp.sqrt(1.0 + 1e-5) + bn2_ref[...])


def kernel(x, qkv_w, qkv_b, mha_out_w, mha_out_b, proj_w, proj_b, mr_w, mr_b,
           fr_w, fr_b, fc1_w, fc1_b, fc2_w, fc2_b, g1, b1, g2, b2):
    f32 = jnp.float32
    x2d = x.reshape(_S, _D)
    mrb = mr_b.reshape(1, _EA)
    frb = fr_b.reshape(1, _EF)
    g1r, b1r = g1.reshape(1, _D), b1.reshape(1, _D)
    g2r, b2r = g2.reshape(1, _D), b2.reshape(1, _D)

    ar, ti, awk = pl.pallas_call(
        _router_a_body,
        out_shape=(jax.ShapeDtypeStruct((1, _EA), f32),
                   jax.ShapeDtypeStruct((1, _K), jnp.int32),
                   jax.ShapeDtypeStruct((1, _K), f32)),
    )(x2d, mr_w, mrb)
    ti1 = ti.reshape(_K)

    qkv = pl.pallas_call(
        _qkv_body,
        grid_spec=pltpu.PrefetchScalarGridSpec(
            num_scalar_prefetch=1,
            grid=(_K, 3),
            in_specs=[
                pl.BlockSpec((_S, _D), lambda e, nb, ti: (0, 0)),
                pl.BlockSpec((1, _D, _D), lambda e, nb, ti: (ti[e], nb, 0)),
                pl.BlockSpec((1, 1, _D), lambda e, nb, ti: (ti[e], 0, nb)),
            ],
            out_specs=pl.BlockSpec((1, _S, _D), lambda e, nb, ti: (e, 0, nb)),
        ),
        out_shape=jax.ShapeDtypeStruct((_K, _S, 3 * _D), f32),
    )(ti1, x2d, qkv_w, qkv_b.reshape(_EA, 1, 3 * _D))

    _HP = _H // 2
    o = pl.pallas_call(
        _attn_body,
        grid=(_K, _HP, _S // _QB),
        in_specs=[
            pl.BlockSpec((1, _QB, 2 * _DH), lambda e, hp, qb: (e, qb, hp)),
            pl.BlockSpec((1, _S, 2 * _DH), lambda e, hp, qb: (e, 0, _HP + hp)),
            pl.BlockSpec((1, _S, 2 * _DH),
                         lambda e, hp, qb: (e, 0, 2 * _HP + hp)),
        ],
        out_specs=pl.BlockSpec((1, _QB, 2 * _DH), lambda e, hp, qb: (e, qb, hp)),
        out_shape=jax.ShapeDtypeStruct((_K, _S, _D), f32),
    )(qkv, qkv, qkv)

    x1 = pl.pallas_call(
        _combine_body,
        grid_spec=pltpu.PrefetchScalarGridSpec(
            num_scalar_prefetch=1,
            grid=(_K, _S // _SB_C),
            in_specs=[
                pl.BlockSpec((1, _SB_C, _D), lambda e, sb, ti: (e, sb, 0)),
                pl.BlockSpec((1, _D, _D), lambda e, sb, ti: (ti[e], 0, 0)),
                pl.BlockSpec((1, 1, _D), lambda e, sb, ti: (ti[e], 0, 0)),
                pl.BlockSpec((1, _D, _D), lambda e, sb, ti: (ti[e], 0, 0)),
                pl.BlockSpec((1, 1, _D), lambda e, sb, ti: (ti[e], 0, 0)),
                pl.BlockSpec((1, _K), lambda e, sb, ti: (0, 0)),
                pl.BlockSpec((_SB_C, _D), lambda e, sb, ti: (sb, 0)),
                pl.BlockSpec((1, _D), lambda e, sb, ti: (0, 0)),
                pl.BlockSpec((1, _D), lambda e, sb, ti: (0, 0)),
            ],
            out_specs=pl.BlockSpec((_SB_C, _D), lambda e, sb, ti: (sb, 0)),
            scratch_shapes=[pltpu.VMEM((_S, _D), f32)],
        ),
        out_shape=jax.ShapeDtypeStruct((_S, _D), f32),
    )(ti1, o, mha_out_w, mha_out_b.reshape(_EA, 1, _D), proj_w,
      proj_b.reshape(_EA, 1, _D), awk, x2d, g1r, b1r)

    fr, r01, w01, te2d = pl.pallas_call(
        _meta_body,
        out_shape=(jax.ShapeDtypeStruct((_S, _EF), f32),
                   jax.ShapeDtypeStruct((_S, 2), f32),
                   jax.ShapeDtypeStruct((_S, 2), f32),
                   jax.ShapeDtypeStruct((_NT, 1), jnp.int32)),
    )(x1, fr_w, frb)
    te = te2d.reshape(_NT)
    r01t = jnp.transpose(r01).reshape(2, 1, _S)

    yg = pl.pallas_call(
        _gffn_body,
        grid_spec=pltpu.PrefetchScalarGridSpec(
            num_scalar_prefetch=1,
            grid=(_NT,),
            in_specs=[
                pl.BlockSpec((1, 1, _S), lambda t, te: (0, 0, 0)),
                pl.BlockSpec((1, 1, _S), lambda t, te: (1, 0, 0)),
                pl.BlockSpec((_S, _D), lambda t, te: (0, 0)),
                pl.BlockSpec((1, _DFF, _D), lambda t, te: (te[t], 0, 0)),
                pl.BlockSpec((1, 1, _DFF), lambda t, te: (te[t], 0, 0)),
                pl.BlockSpec((1, _D, _DFF), lambda t, te: (te[t], 0, 0)),
                pl.BlockSpec((1, 1, _D), lambda t, te: (te[t], 0, 0)),
            ],
            out_specs=pl.BlockSpec((_TILE, _D), lambda t, te: (t, 0)),
        ),
        out_shape=jax.ShapeDtypeStruct((_NROW, _D), f32),
    )(te, r01t, r01t, x1, fc1_w, fc1_b.reshape(_EF, 1, _DFF), fc2_w,
      fc2_b.reshape(_EF, 1, _D))

    x2 = pl.pallas_call(
        _fcomb_body,
        grid=(_S // _SB_F,),
        in_specs=[
            pl.BlockSpec((_NROW, _D), lambda sb: (0, 0)),
            pl.BlockSpec((_SB_F, 2), lambda sb: (sb, 0)),
            pl.BlockSpec((_SB_F, 2), lambda sb: (sb, 0)),
            pl.BlockSpec((_SB_F, _D), lambda sb: (sb, 0)),
            pl.BlockSpec((1, _D), lambda sb: (0, 0)),
            pl.BlockSpec((1, _D), lambda sb: (0, 0)),
        ],
        out_specs=pl.BlockSpec((_SB_F, _D), lambda sb: (sb, 0)),
        out_shape=jax.ShapeDtypeStruct((_S, _D), f32),
    )(yg, r01, w01, x1, g2r, b2r)

    return (x2.reshape(_B, _S, _D), ar.reshape(_EA), fr)


# speedup vs baseline: 1.0034x; 1.0034x over previous
import jax
import jax.numpy as jnp
from jax.experimental import pallas as pl
from jax.experimental.pallas import tpu as pltpu

_B, _S, _D, _H = 1, 2048, 768, 12
_EA, _EF, _K = 8, 8, 2
_DH = _D // _H
_DFF = 4 * _D
_SB_C = 1024
_SB_F = 512
_QB = 2048


def _mm_t(a, b):
    return jax.lax.dot_general(a, b, (((1,), (1,)), ((), ())),
                               preferred_element_type=jnp.float32)


def _router_a_body(x_ref, mrw_ref, mrb_ref, ar_ref, ti_ref, awk_ref):
    xm = jnp.mean(x_ref[...], axis=0, keepdims=True)
    logits = jnp.dot(xm, mrw_ref[...],
                     preferred_element_type=jnp.float32) + mrb_ref[...]
    ar = jax.nn.softmax(logits, axis=-1)
    iota = jax.lax.broadcasted_iota(jnp.int32, (1, _EA), 1)
    i1 = jnp.argmax(ar, axis=-1)
    m1 = iota == i1[:, None]
    i2 = jnp.argmax(jnp.where(m1, -1.0, ar), axis=-1)
    m2 = iota == i2[:, None]
    mask = (m1 | m2).astype(jnp.float32)
    aw = ar * mask
    aw = aw / (jnp.sum(aw, axis=-1, keepdims=True) + 1e-9)
    w1 = jnp.sum(jnp.where(m1, aw, 0.0), axis=-1)
    w2 = jnp.sum(jnp.where(m2, aw, 0.0), axis=-1)
    ar_ref[...] = ar
    ti_ref[...] = jnp.stack([i1, i2], axis=-1).astype(jnp.int32)
    awk_ref[...] = jnp.stack([w1, w2], axis=-1)


def _qkv_body(ti_ref, x_ref, w_ref, b_ref, out_ref):
    del ti_ref
    y = _mm_t(x_ref[...], w_ref[0]) + b_ref[0]
    out_ref[0] = y


def _attn_body(q_ref, k_ref, v_ref, o_ref):
    log2e_over_sqrt_dh = 1.4426950408889634 / 8.0
    outs = []
    for hh in range(2):
        q = q_ref[0][:, hh * _DH:(hh + 1) * _DH] * log2e_over_sqrt_dh
        k = k_ref[0][:, hh * _DH:(hh + 1) * _DH]
        v = v_ref[0][:, hh * _DH:(hh + 1) * _DH]
        s2 = _mm_t(q, k)
        m = jnp.max(s2, axis=-1, keepdims=True)
        e2 = jnp.exp2(s2 - m)
        u = jnp.dot(e2, v, preferred_element_type=jnp.float32)
        r = 1.0 / jnp.sum(e2, axis=-1, keepdims=True)
        outs.append(u * r)
    o_ref[0] = jnp.concatenate(outs, axis=1)


def _combine_body(ti_ref, o_ref, wo_ref, bo_ref, wp_ref, bp_ref, awk_ref,
                  x_ref, g1_ref, b1_ref, out_ref, acc_ref):
    del ti_ref
    e = pl.program_id(0)
    sb = pl.program_id(1)
    t = _mm_t(o_ref[0], wo_ref[0]) + bo_ref[0]
    r = _mm_t(t, wp_ref[0]) + bp_ref[0]
    w_e = jnp.sum(jnp.where(
        jax.lax.broadcasted_iota(jnp.int32, (1, _K), 1) == e,
        awk_ref[...], 0.0))
    c = w_e * r
    sl = pl.ds(sb * _SB_C, _SB_C)

    @pl.when(e == 0)
    def _():
        acc_ref[sl, :] = c

    @pl.when(e == 1)
    def _():
        attn = acc_ref[sl, :] + c
        out_ref[...] = (g1_ref[...] * (x_ref[...] + attn)
                        / jnp.sqrt(1.0 + 1e-5) + b1_ref[...])


_TILE = 256
_NROW = -(-(2 * _S + _EF * (_TILE - 1)) // _TILE) * _TILE
_NT = _NROW // _TILE


def _meta_body(x1_ref, frw_ref, frb_ref, fr_ref, r01_ref, w01_ref, te_ref):
    logits = jnp.dot(x1_ref[...], frw_ref[...],
                     preferred_element_type=jnp.float32) + frb_ref[...]
    fr = jax.nn.softmax(logits, axis=-1)
    iota = jax.lax.broadcasted_iota(jnp.int32, (_S, _EF), 1)
    i1 = jnp.argmax(fr, axis=-1)
    m1 = iota == i1[:, None]
    i2 = jnp.argmax(jnp.where(m1, -1.0, fr), axis=-1)
    m2 = iota == i2[:, None]
    fwm = fr * (m1 | m2).astype(jnp.float32)
    fw = fwm / (jnp.sum(fwm, axis=-1, keepdims=True) + 1e-9)
    fr_ref[...] = fr

    m1f = m1.astype(jnp.float32)
    m2f = m2.astype(jnp.float32)
    rio = jax.lax.broadcasted_iota(jnp.int32, (_S, 1), 0)
    cio = jax.lax.broadcasted_iota(jnp.int32, (1, _S), 1)
    tri = (cio <= rio).astype(jnp.bfloat16)
    cs1 = jnp.dot(tri, m1.astype(jnp.bfloat16),
                  preferred_element_type=jnp.float32)
    cs2 = jnp.dot(tri, m2.astype(jnp.bfloat16),
                  preferred_element_type=jnp.float32)
    tot1 = cs1[_S - 1:_S, :]
    counts = tot1 + cs2[_S - 1:_S, :]
    padded = jnp.floor((counts + (_TILE - 1)) * (1.0 / _TILE)) * _TILE
    e8r = jax.lax.broadcasted_iota(jnp.int32, (_EF, _EF), 0)
    e8c = jax.lax.broadcasted_iota(jnp.int32, (_EF, _EF), 1)
    t8 = (e8r < e8c).astype(jnp.float32)
    offs = jnp.dot(padded, t8, preferred_element_type=jnp.float32)
    r0 = jnp.sum(m1f * (cs1 - 1.0 + offs), axis=1, keepdims=True)
    r1 = jnp.sum(m2f * (cs2 - 1.0 + offs + tot1), axis=1, keepdims=True)
    r01_ref[...] = jnp.concatenate([r0, r1], axis=1)
    w0 = jnp.sum(fw * m1f, axis=1, keepdims=True)
    w1 = jnp.sum(fw * m2f, axis=1, keepdims=True)
    w01_ref[...] = jnp.concatenate([w0, w1], axis=1)
    tpos = (jax.lax.broadcasted_iota(jnp.int32, (_NT, 1), 0)
            .astype(jnp.float32) * _TILE)
    te = jnp.sum((offs <= tpos).astype(jnp.int32), axis=1,
                 keepdims=True) - 1
    te_ref[...] = te


def _gffn_body(te_ref, r0_ref, r1_ref, x1_ref, w1_ref, b1_ref, w2_ref,
               b2_ref, yg_ref):
    del te_ref
    t = pl.program_id(0)
    rowio = (jax.lax.broadcasted_iota(jnp.int32, (_TILE, 1), 0)
             .astype(jnp.float32) + jnp.float32(_TILE) * t)
    r0v = r0_ref[0]
    r1v = r1_ref[0]
    p = ((r0v == rowio) | (r1v == rowio)).astype(jnp.float32)
    xg = jnp.dot(p, x1_ref[...], preferred_element_type=jnp.float32)
    z = _mm_t(xg, w1_ref[0]) + b1_ref[0]
    h = 0.5 * z * (1.0 + jax.lax.erf(z * (2.0 ** -0.5)))
    yg_ref[...] = _mm_t(h, w2_ref[0]) + b2_ref[0]


def _fcomb_body(yg_ref, r01_ref, w01_ref, x1_ref, g2_ref, bn2_ref, out_ref):
    r0c = r01_ref[:, 0:1]
    r1c = r01_ref[:, 1:2]
    w0c = w01_ref[:, 0:1]
    w1c = w01_ref[:, 1:2]
    colio = (jax.lax.broadcasted_iota(jnp.int32, (1, _NROW), 1)
             .astype(jnp.float32))
    g = (jnp.where(r0c == colio, w0c, 0.0)
         + jnp.where(r1c == colio, w1c, 0.0))
    acc = jnp.dot(g, yg_ref[...], preferred_element_type=jnp.float32)
    out_ref[...] = (g2_ref[...] * (x1_ref[...] + acc)
                    / jnp.sqrt(1.0 + 1e-5) + bn2_ref[...])


def kernel(x, qkv_w, qkv_b, mha_out_w, mha_out_b, proj_w, proj_b, mr_w, mr_b,
           fr_w, fr_b, fc1_w, fc1_b, fc2_w, fc2_b, g1, b1, g2, b2):
    f32 = jnp.float32
    x2d = x.reshape(_S, _D)
    mrb = mr_b.reshape(1, _EA)
    frb = fr_b.reshape(1, _EF)
    g1r, b1r = g1.reshape(1, _D), b1.reshape(1, _D)
    g2r, b2r = g2.reshape(1, _D), b2.reshape(1, _D)

    ar, ti, awk = pl.pallas_call(
        _router_a_body,
        out_shape=(jax.ShapeDtypeStruct((1, _EA), f32),
                   jax.ShapeDtypeStruct((1, _K), jnp.int32),
                   jax.ShapeDtypeStruct((1, _K), f32)),
    )(x2d, mr_w, mrb)
    ti1 = ti.reshape(_K)

    qkv = pl.pallas_call(
        _qkv_body,
        grid_spec=pltpu.PrefetchScalarGridSpec(
            num_scalar_prefetch=1,
            grid=(_K, 3),
            in_specs=[
                pl.BlockSpec((_S, _D), lambda e, nb, ti: (0, 0)),
                pl.BlockSpec((1, _D, _D), lambda e, nb, ti: (ti[e], nb, 0)),
                pl.BlockSpec((1, 1, _D), lambda e, nb, ti: (ti[e], 0, nb)),
            ],
            out_specs=pl.BlockSpec((1, _S, _D), lambda e, nb, ti: (e, 0, nb)),
        ),
        out_shape=jax.ShapeDtypeStruct((_K, _S, 3 * _D), f32),
    )(ti1, x2d, qkv_w, qkv_b.reshape(_EA, 1, 3 * _D))

    _HP = _H // 2
    o = pl.pallas_call(
        _attn_body,
        grid=(_K, _HP, _S // _QB),
        in_specs=[
            pl.BlockSpec((1, _QB, 2 * _DH), lambda e, hp, qb: (e, qb, hp)),
            pl.BlockSpec((1, _S, 2 * _DH), lambda e, hp, qb: (e, 0, _HP + hp)),
            pl.BlockSpec((1, _S, 2 * _DH),
                         lambda e, hp, qb: (e, 0, 2 * _HP + hp)),
        ],
        out_specs=pl.BlockSpec((1, _QB, 2 * _DH), lambda e, hp, qb: (e, qb, hp)),
        out_shape=jax.ShapeDtypeStruct((_K, _S, _D), f32),
    )(qkv, qkv, qkv)

    x1 = pl.pallas_call(
        _combine_body,
        grid_spec=pltpu.PrefetchScalarGridSpec(
            num_scalar_prefetch=1,
            grid=(_K, _S // _SB_C),
            in_specs=[
                pl.BlockSpec((1, _SB_C, _D), lambda e, sb, ti: (e, sb, 0)),
                pl.BlockSpec((1, _D, _D), lambda e, sb, ti: (ti[e], 0, 0)),
                pl.BlockSpec((1, 1, _D), lambda e, sb, ti: (ti[e], 0, 0)),
                pl.BlockSpec((1, _D, _D), lambda e, sb, ti: (ti[e], 0, 0)),
                pl.BlockSpec((1, 1, _D), lambda e, sb, ti: (ti[e], 0, 0)),
                pl.BlockSpec((1, _K), lambda e, sb, ti: (0, 0)),
                pl.BlockSpec((_SB_C, _D), lambda e, sb, ti: (sb, 0)),
                pl.BlockSpec((1, _D), lambda e, sb, ti: (0, 0)),
                pl.BlockSpec((1, _D), lambda e, sb, ti: (0, 0)),
            ],
            out_specs=pl.BlockSpec((_SB_C, _D), lambda e, sb, ti: (sb, 0)),
            scratch_shapes=[pltpu.VMEM((_S, _D), f32)],
        ),
        out_shape=jax.ShapeDtypeStruct((_S, _D), f32),
    )(ti1, o, mha_out_w, mha_out_b.reshape(_EA, 1, _D), proj_w,
      proj_b.reshape(_EA, 1, _D), awk, x2d, g1r, b1r)

    fr, r01, w01, te2d = pl.pallas_call(
        _meta_body,
        out_shape=(jax.ShapeDtypeStruct((_S, _EF), f32),
                   jax.ShapeDtypeStruct((_S, 2), f32),
                   jax.ShapeDtypeStruct((_S, 2), f32),
                   jax.ShapeDtypeStruct((_NT, 1), jnp.int32)),
    )(x1, fr_w, frb)
    te = te2d.reshape(_NT)
    r01t = jnp.transpose(r01).reshape(2, 1, _S)

    yg = pl.pallas_call(
        _gffn_body,
        grid_spec=pltpu.PrefetchScalarGridSpec(
            num_scalar_prefetch=1,
            grid=(_NT,),
            in_specs=[
                pl.BlockSpec((1, 1, _S), lambda t, te: (0, 0, 0)),
                pl.BlockSpec((1, 1, _S), lambda t, te: (1, 0, 0)),
                pl.BlockSpec((_S, _D), lambda t, te: (0, 0)),
                pl.BlockSpec((1, _DFF, _D), lambda t, te: (te[t], 0, 0)),
                pl.BlockSpec((1, 1, _DFF), lambda t, te: (te[t], 0, 0)),
                pl.BlockSpec((1, _D, _DFF), lambda t, te: (te[t], 0, 0)),
                pl.BlockSpec((1, 1, _D), lambda t, te: (te[t], 0, 0)),
            ],
            out_specs=pl.BlockSpec((_TILE, _D), lambda t, te: (t, 0)),
        ),
        out_shape=jax.ShapeDtypeStruct((_NROW, _D), f32),
    )(te, r01t, r01t, x1, fc1_w, fc1_b.reshape(_EF, 1, _DFF), fc2_w,
      fc2_b.reshape(_EF, 1, _D))

    x2 = pl.pallas_call(
        _fcomb_body,
        grid=(_S // _SB_F,),
        in_specs=[
            pl.BlockSpec((_NROW, _D), lambda sb: (0, 0)),
            pl.BlockSpec((_SB_F, 2), lambda sb: (sb, 0)),
            pl.BlockSpec((_SB_F, 2), lambda sb: (sb, 0)),
            pl.BlockSpec((_SB_F, _D), lambda sb: (sb, 0)),
            pl.BlockSpec((1, _D), lambda sb: (0, 0)),
            pl.BlockSpec((1, _D), lambda sb: (0, 0)),
        ],
        out_specs=pl.BlockSpec((_SB_F, _D), lambda sb: (sb, 0)),
        out_shape=jax.ShapeDtypeStruct((_S, _D), f32),
    )(yg, r01, w01, x1, g2r, b2r)

    return (x2.reshape(_B, _S, _D), ar.reshape(_EA), fr)
